# Initial kernel scaffold; baseline (speedup 1.0000x reference)
import jax
import jax.numpy as jnp
from jax.experimental import pallas as pl


def kernel(x, edge_index, edge_weight, W1, b1, W2, b2, lin_w, lin_b, bn1_w, bn1_b, bn1_rm, bn1_rv, bn2_w, bn2_b, bn2_rm, bn2_rv):
    raise NotImplementedError("write your pallas kernel here")


# speedup vs baseline: 8.2339x; 8.2339x over previous
import functools

import jax
import jax.numpy as jnp
from jax import lax
from jax.experimental import pallas as pl
from jax.experimental.pallas import tpu as pltpu
from jax.experimental.pallas import tpu_sc as plsc

N = 10000
E = 320000
IN_F = 128
CL1 = 256
CL2 = 256
OUT_F = 64
BN_EPS = 1e-5

HALF = 128
CHUNK = 128
NCHUNKS = E // CHUNK
NTILES = 16
N_PAD = 10240
STRIPE = N_PAD // NTILES

_f32 = jnp.float32


def _deg_body(dst2, ew2, deg_a, deg_b, acc_s, dst_v, ew_v, zbuf):
    cid = lax.axis_index("c")
    s = lax.axis_index("s")

    @pl.when(s == 0)
    def _():
        def zr(i, _):
            zbuf[pl.ds(i * 16, 16)] = jnp.zeros((16,), _f32)
            return 0
        lax.fori_loop(0, 125, zr, 0)
        for k in range(5):
            pltpu.sync_copy(zbuf, acc_s.at[pl.ds(k * 2000, 2000)])

    plsc.subcore_barrier()

    start = cid * 1250 + jnp.where(s < 2, s * 79, 158 + (s - 2) * 78)
    cnt = jnp.where(s < 2, 79, 78)

    def cb(j, _):
        pltpu.sync_copy(dst2.at[j], dst_v)
        pltpu.sync_copy(ew2.at[j], ew_v)
        pltpu.sync_copy(ew_v, acc_s.at[dst_v], add=True)
        return 0
    lax.fori_loop(start, start + cnt, cb, 0)

    plsc.subcore_barrier()

    @pl.when((s == 0) & (cid == 0))
    def _():
        pltpu.sync_copy(acc_s, deg_a)

    @pl.when((s == 0) & (cid == 1))
    def _():
        pltpu.sync_copy(acc_s, deg_b)


def _deg_call(dst2, ew2):
    mesh = plsc.VectorSubcoreMesh(core_axis_name="c", subcore_axis_name="s")
    return pl.kernel(
        _deg_body,
        out_type=[jax.ShapeDtypeStruct((N,), _f32),
                  jax.ShapeDtypeStruct((N,), _f32)],
        mesh=mesh,
        scratch_types=[
            pltpu.VMEM_SHARED((N,), _f32),
            pltpu.VMEM((CHUNK,), jnp.int32),
            pltpu.VMEM((CHUNK,), _f32),
            pltpu.VMEM((2000,), _f32),
        ],
    )(dst2, ew2)


def _prop_half(src2, dst2, ew2, h_ref, out_ref, acc_s, src_v, dst_v, ew_v,
               rows_v, zbuf, sem):
    s = lax.axis_index("s")

    def zrow(i, _):
        for f in range(8):
            zbuf[i, pl.ds(f * 16, 16)] = jnp.zeros((16,), _f32)
        return 0
    lax.fori_loop(0, 128, zrow, 0)
    row0 = s * STRIPE
    for k in range(5):
        pltpu.sync_copy(zbuf, acc_s.at[pl.ds(row0 + k * 128, 128)])

    plsc.subcore_barrier()

    start = jnp.where(s < 2, s * 158, 316 + (s - 2) * 156)
    cnt = jnp.where(s < 2, 158, 156)

    def chunk_body(j, _):
        pltpu.sync_copy(src2.at[j], src_v)
        pltpu.sync_copy(dst2.at[j], dst_v)
        pltpu.sync_copy(ew2.at[j], ew_v)
        pltpu.async_copy(h_ref.at[src_v], rows_v, sem).wait()

        def group_body(g, _):
            wvec = ew_v[pl.ds(g * 16, 16)]
            for l in range(16):
                e = g * 16 + l
                w = jnp.full((16,), wvec[l], _f32)
                for f in range(8):
                    sl = pl.ds(f * 16, 16)
                    rows_v[e, sl] = rows_v[e, sl] * w
            return 0
        lax.fori_loop(0, CHUNK // 16, group_body, 0)

        pltpu.sync_copy(rows_v, acc_s.at[dst_v], add=True)
        return 0
    lax.fori_loop(start, start + cnt, chunk_body, 0)

    plsc.subcore_barrier()

    for k in range(5):
        sl = pl.ds(row0 + k * 128, 128)
        pltpu.sync_copy(acc_s.at[sl], out_ref.at[sl])


def _prop_body(src2, dst2, ew2, h_lo, h_hi, out_lo, out_hi, acc_s, src_v,
               dst_v, ew_v, rows_v, zbuf, sem):
    cid = lax.axis_index("c")

    @pl.when(cid == 0)
    def _():
        _prop_half(src2, dst2, ew2, h_lo, out_lo, acc_s, src_v, dst_v, ew_v,
                   rows_v, zbuf, sem)

    @pl.when(cid == 1)
    def _():
        _prop_half(src2, dst2, ew2, h_hi, out_hi, acc_s, src_v, dst_v, ew_v,
                   rows_v, zbuf, sem)


def _prop_call(src2, dst2, ew2, h_lo, h_hi):
    mesh = plsc.VectorSubcoreMesh(core_axis_name="c", subcore_axis_name="s")
    return pl.kernel(
        _prop_body,
        out_type=[jax.ShapeDtypeStruct((N_PAD, HALF), _f32),
                  jax.ShapeDtypeStruct((N_PAD, HALF), _f32)],
        mesh=mesh,
        scratch_types=[
            pltpu.VMEM_SHARED((N_PAD, HALF), _f32),
            pltpu.VMEM((CHUNK,), jnp.int32),
            pltpu.VMEM((CHUNK,), jnp.int32),
            pltpu.VMEM((CHUNK,), _f32),
            pltpu.VMEM((CHUNK, HALF), _f32),
            pltpu.VMEM((CHUNK, HALF), _f32),
            pltpu.SemaphoreType.DMA,
        ],
    )(src2, dst2, ew2, h_lo, h_hi)


ROWB = 1000


def _tc1_body(dega, degb, x, W1, dis_o, hlo_o, hhi_o):
    deg = dega[...] + degb[...] + 1.0
    dis = lax.rsqrt(deg)
    h = lax.dot_general(x[...], W1[...], (((1,), (1,)), ((), ())),
                        preferred_element_type=_f32)
    h = h * dis
    dis_o[...] = dis
    hlo_o[...] = h[:, :HALF]
    hhi_o[...] = h[:, HALF:]


def _tc1_call(dega, degb, x, W1):
    return pl.pallas_call(
        _tc1_body,
        grid=(N // ROWB,),
        in_specs=[
            pl.BlockSpec((ROWB, 1), lambda i: (i, 0)),
            pl.BlockSpec((ROWB, 1), lambda i: (i, 0)),
            pl.BlockSpec((ROWB, IN_F), lambda i: (i, 0)),
            pl.BlockSpec((CL1, IN_F), lambda i: (0, 0)),
        ],
        out_specs=[
            pl.BlockSpec((ROWB, 1), lambda i: (i, 0)),
            pl.BlockSpec((ROWB, HALF), lambda i: (i, 0)),
            pl.BlockSpec((ROWB, HALF), lambda i: (i, 0)),
        ],
        out_shape=[jax.ShapeDtypeStruct((N, 1), _f32),
                   jax.ShapeDtypeStruct((N, HALF), _f32),
                   jax.ShapeDtypeStruct((N, HALF), _f32)],
    )(dega, degb, x, W1)


def _tc2_body(acclo, acchi, hlo, hhi, dis, b, bw, bb, brm, brv, W2,
              h2lo_o, h2hi_o):
    z = jnp.concatenate([acclo[...] + hlo[...], acchi[...] + hhi[...]], axis=1)
    d = dis[...]
    z = jax.nn.relu(z * d + b[...])
    sc = bw[...] * lax.rsqrt(brv[...] + BN_EPS)
    z = (z - brm[...]) * sc + bb[...]
    h2 = lax.dot_general(z, W2[...], (((1,), (1,)), ((), ())),
                         preferred_element_type=_f32)
    h2 = h2 * d
    h2lo_o[...] = h2[:, :HALF]
    h2hi_o[...] = h2[:, HALF:]


def _tc2_call(acclo, acchi, hlo, hhi, dis, b1, bn_w, bn_b, bn_rm, bn_rv, W2):
    row = lambda i: (i, 0)
    fixed = lambda i: (0, 0)
    return pl.pallas_call(
        _tc2_body,
        grid=(N // ROWB,),
        in_specs=[
            pl.BlockSpec((ROWB, HALF), row),
            pl.BlockSpec((ROWB, HALF), row),
            pl.BlockSpec((ROWB, HALF), row),
            pl.BlockSpec((ROWB, HALF), row),
            pl.BlockSpec((ROWB, 1), row),
            pl.BlockSpec((1, CL1), fixed),
            pl.BlockSpec((1, CL1), fixed),
            pl.BlockSpec((1, CL1), fixed),
            pl.BlockSpec((1, CL1), fixed),
            pl.BlockSpec((1, CL1), fixed),
            pl.BlockSpec((CL2, CL1), fixed),
        ],
        out_specs=[
            pl.BlockSpec((ROWB, HALF), row),
            pl.BlockSpec((ROWB, HALF), row),
        ],
        out_shape=[jax.ShapeDtypeStruct((N, HALF), _f32),
                   jax.ShapeDtypeStruct((N, HALF), _f32)],
    )(acclo, acchi, hlo, hhi, dis, b1, bn_w, bn_b, bn_rm, bn_rv, W2)


def _tc3_body(acclo, acchi, hlo, hhi, dis, b, bw, bb, brm, brv, lw, lb, out_o):
    z = jnp.concatenate([acclo[...] + hlo[...], acchi[...] + hhi[...]], axis=1)
    z = jax.nn.relu(z * dis[...] + b[...])
    sc = bw[...] * lax.rsqrt(brv[...] + BN_EPS)
    z = (z - brm[...]) * sc + bb[...]
    out = lax.dot_general(z, lw[...], (((1,), (1,)), ((), ())),
                          preferred_element_type=_f32)
    out_o[...] = out + lb[...]


def _tc3_call(acclo, acchi, hlo, hhi, dis, b2, bn_w, bn_b, bn_rm, bn_rv,
              lin_w, lin_b):
    row = lambda i: (i, 0)
    fixed = lambda i: (0, 0)
    return pl.pallas_call(
        _tc3_body,
        grid=(N // ROWB,),
        in_specs=[
            pl.BlockSpec((ROWB, HALF), row),
            pl.BlockSpec((ROWB, HALF), row),
            pl.BlockSpec((ROWB, HALF), row),
            pl.BlockSpec((ROWB, HALF), row),
            pl.BlockSpec((ROWB, 1), row),
            pl.BlockSpec((1, CL2), fixed),
            pl.BlockSpec((1, CL2), fixed),
            pl.BlockSpec((1, CL2), fixed),
            pl.BlockSpec((1, CL2), fixed),
            pl.BlockSpec((1, CL2), fixed),
            pl.BlockSpec((OUT_F, CL2), fixed),
            pl.BlockSpec((1, OUT_F), fixed),
        ],
        out_specs=pl.BlockSpec((ROWB, OUT_F), row),
        out_shape=jax.ShapeDtypeStruct((N, OUT_F), _f32),
    )(acclo, acchi, hlo, hhi, dis, b2, bn_w, bn_b, bn_rm, bn_rv, lin_w, lin_b)


def kernel(x, edge_index, edge_weight, W1, b1, W2, b2, lin_w, lin_b,
           bn1_w, bn1_b, bn1_rm, bn1_rv, bn2_w, bn2_b, bn2_rm, bn2_rv):
    src2 = edge_index[0].astype(jnp.int32).reshape(NCHUNKS, CHUNK)
    dst2 = edge_index[1].astype(jnp.int32).reshape(NCHUNKS, CHUNK)
    ew2 = edge_weight.astype(_f32).reshape(NCHUNKS, CHUNK)

    deg_a, deg_b = _deg_call(dst2, ew2)
    dega = deg_a.reshape(N, 1)
    degb = deg_b.reshape(N, 1)

    dis, h1lo, h1hi = _tc1_call(dega, degb, x, W1)
    acc1lo, acc1hi = _prop_call(src2, dst2, ew2, h1lo, h1hi)
    acc1lo, acc1hi = acc1lo[:N], acc1hi[:N]
    h2lo, h2hi = _tc2_call(acc1lo, acc1hi, h1lo, h1hi, dis,
                           b1.reshape(1, CL1), bn1_w.reshape(1, CL1),
                           bn1_b.reshape(1, CL1), bn1_rm.reshape(1, CL1),
                           bn1_rv.reshape(1, CL1), W2)
    acc2lo, acc2hi = _prop_call(src2, dst2, ew2, h2lo, h2hi)
    acc2lo, acc2hi = acc2lo[:N], acc2hi[:N]
    out = _tc3_call(acc2lo, acc2hi, h2lo, h2hi, dis,
                    b2.reshape(1, CL2), bn2_w.reshape(1, CL2),
                    bn2_b.reshape(1, CL2), bn2_rm.reshape(1, CL2),
                    bn2_rv.reshape(1, CL2), lin_w, lin_b.reshape(1, OUT_F))
    return out


# speedup vs baseline: 9.0386x; 1.0977x over previous
import functools

import jax
import jax.numpy as jnp
from jax import lax
from jax.experimental import pallas as pl
from jax.experimental.pallas import tpu as pltpu
from jax.experimental.pallas import tpu_sc as plsc

N = 10000
E = 320000
IN_F = 128
CL1 = 256
CL2 = 256
OUT_F = 64
BN_EPS = 1e-5

HALF = 128
CHUNK = 128
NTILES = 16
NCH_TILE = 160
NCH_PAD = NTILES * NCH_TILE
E_PAD = NCH_PAD * CHUNK
NCH_DEG = NCH_PAD // 32
PH = 40
N_PAD = 10240
STRIPE = N_PAD // NTILES

_f32 = jnp.float32


def _deg_body(dst2, ew2, deg_a, deg_b, acc_s, dst_all, ew_all, zbuf):
    cid = lax.axis_index("c")
    s = lax.axis_index("s")

    @pl.when(s == 0)
    def _():
        def zr(i, _):
            zbuf[pl.ds(i * 16, 16)] = jnp.zeros((16,), _f32)
            return 0
        lax.fori_loop(0, 125, zr, 0)
        for k in range(5):
            pltpu.sync_copy(zbuf, acc_s.at[pl.ds(k * 2000, 2000)])

    start = (cid * NTILES + s) * NCH_DEG
    pltpu.sync_copy(dst2.at[pl.ds(start, NCH_DEG)], dst_all)
    pltpu.sync_copy(ew2.at[pl.ds(start, NCH_DEG)], ew_all)

    plsc.subcore_barrier()

    def cb(j, _):
        pltpu.sync_copy(ew_all.at[j], acc_s.at[dst_all.at[j]], add=True)
        return 0
    lax.fori_loop(0, NCH_DEG, cb, 0)

    plsc.subcore_barrier()

    @pl.when((s == 0) & (cid == 0))
    def _():
        pltpu.sync_copy(acc_s, deg_a)

    @pl.when((s == 0) & (cid == 1))
    def _():
        pltpu.sync_copy(acc_s, deg_b)


def _deg_call(dst2, ew2):
    mesh = plsc.VectorSubcoreMesh(core_axis_name="c", subcore_axis_name="s")
    return pl.kernel(
        _deg_body,
        out_type=[jax.ShapeDtypeStruct((N,), _f32),
                  jax.ShapeDtypeStruct((N,), _f32)],
        mesh=mesh,
        scratch_types=[
            pltpu.VMEM_SHARED((N,), _f32),
            pltpu.VMEM((NCH_DEG, CHUNK), jnp.int32),
            pltpu.VMEM((NCH_DEG, CHUNK), _f32),
            pltpu.VMEM((2000,), _f32),
        ],
    )(dst2, ew2)


def _scale_rows(rows_v, ew_all, j):
    def group_body(g, _):
        wvec = ew_all[j, pl.ds(g * 16, 16)]
        for l in range(16):
            e = g * 16 + l
            w = jnp.full((16,), wvec[l], _f32)
            for f in range(8):
                sl = pl.ds(f * 16, 16)
                rows_v[e, sl] = rows_v[e, sl] * w
        return 0
    lax.fori_loop(0, CHUNK // 16, group_body, 0)


def _prop_half(src2, dst2, ew2, h_ref, out_ref, acc_s, src_all, dst_all,
               ew_all, rows_a, rows_b, sem):
    s = lax.axis_index("s")

    def zrow(i, _):
        for f in range(8):
            rows_a[i, pl.ds(f * 16, 16)] = jnp.zeros((16,), _f32)
        return 0
    lax.fori_loop(0, CHUNK, zrow, 0)
    row0 = s * STRIPE
    for k in range(5):
        pltpu.sync_copy(rows_a, acc_s.at[pl.ds(row0 + k * 128, 128)])

    plsc.subcore_barrier()

    def phase_body(p, _):
        base = s * NCH_TILE + p * PH
        pltpu.sync_copy(src2.at[pl.ds(base, PH)], src_all)
        pltpu.sync_copy(dst2.at[pl.ds(base, PH)], dst_all)
        pltpu.sync_copy(ew2.at[pl.ds(base, PH)], ew_all)
        pltpu.async_copy(h_ref.at[src_all.at[0]], rows_a, sem)

        def pair_body(i, _):
            for b in range(2):
                j = 2 * i + b
                cur, nxt = (rows_a, rows_b) if b == 0 else (rows_b, rows_a)
                pltpu.make_async_copy(h_ref.at[src_all.at[j]], cur,
                                      sem).wait()
                jn = jnp.minimum(j + 1, PH - 1)
                pltpu.async_copy(h_ref.at[src_all.at[jn]], nxt, sem)
                _scale_rows(cur, ew_all, j)
                pltpu.sync_copy(cur, acc_s.at[dst_all.at[j]], add=True)
            return 0
        lax.fori_loop(0, PH // 2, pair_body, 0)
        pltpu.make_async_copy(h_ref.at[src_all.at[PH - 1]], rows_a,
                              sem).wait()
        return 0
    lax.fori_loop(0, NCH_TILE // PH, phase_body, 0)

    plsc.subcore_barrier()

    for k in range(5):
        sl = pl.ds(row0 + k * 128, 128)
        pltpu.sync_copy(acc_s.at[sl], out_ref.at[sl])


def _prop_body(src2, dst2, ew2, h_lo, h_hi, out_lo, out_hi, acc_s, src_all,
               dst_all, ew_all, rows_a, rows_b, sem):
    cid = lax.axis_index("c")

    @pl.when(cid == 0)
    def _():
        _prop_half(src2, dst2, ew2, h_lo, out_lo, acc_s, src_all, dst_all,
                   ew_all, rows_a, rows_b, sem)

    @pl.when(cid == 1)
    def _():
        _prop_half(src2, dst2, ew2, h_hi, out_hi, acc_s, src_all, dst_all,
                   ew_all, rows_a, rows_b, sem)


def _prop_call(src2, dst2, ew2, h_lo, h_hi):
    mesh = plsc.VectorSubcoreMesh(core_axis_name="c", subcore_axis_name="s")
    return pl.kernel(
        _prop_body,
        out_type=[jax.ShapeDtypeStruct((N_PAD, HALF), _f32),
                  jax.ShapeDtypeStruct((N_PAD, HALF), _f32)],
        mesh=mesh,
        scratch_types=[
            pltpu.VMEM_SHARED((N_PAD, HALF), _f32),
            pltpu.VMEM((PH, CHUNK), jnp.int32),
            pltpu.VMEM((PH, CHUNK), jnp.int32),
            pltpu.VMEM((PH, CHUNK), _f32),
            pltpu.VMEM((CHUNK, HALF), _f32),
            pltpu.VMEM((CHUNK, HALF), _f32),
            pltpu.SemaphoreType.DMA,
        ],
    )(src2, dst2, ew2, h_lo, h_hi)


ROWB = 1000


def _tc1_body(dega, degb, x, W1, dis_o, hlo_o, hhi_o):
    deg = dega[...] + degb[...] + 1.0
    dis = lax.rsqrt(deg)
    h = lax.dot_general(x[...], W1[...], (((1,), (1,)), ((), ())),
                        preferred_element_type=_f32)
    h = h * dis
    dis_o[...] = dis
    hlo_o[...] = h[:, :HALF]
    hhi_o[...] = h[:, HALF:]


def _tc1_call(dega, degb, x, W1):
    return pl.pallas_call(
        _tc1_body,
        grid=(N // ROWB,),
        in_specs=[
            pl.BlockSpec((ROWB, 1), lambda i: (i, 0)),
            pl.BlockSpec((ROWB, 1), lambda i: (i, 0)),
            pl.BlockSpec((ROWB, IN_F), lambda i: (i, 0)),
            pl.BlockSpec((CL1, IN_F), lambda i: (0, 0)),
        ],
        out_specs=[
            pl.BlockSpec((ROWB, 1), lambda i: (i, 0)),
            pl.BlockSpec((ROWB, HALF), lambda i: (i, 0)),
            pl.BlockSpec((ROWB, HALF), lambda i: (i, 0)),
        ],
        out_shape=[jax.ShapeDtypeStruct((N, 1), _f32),
                   jax.ShapeDtypeStruct((N, HALF), _f32),
                   jax.ShapeDtypeStruct((N, HALF), _f32)],
    )(dega, degb, x, W1)


def _tc2_body(acclo, acchi, hlo, hhi, dis, b, bw, bb, brm, brv, W2,
              h2lo_o, h2hi_o):
    z = jnp.concatenate([acclo[...] + hlo[...], acchi[...] + hhi[...]], axis=1)
    d = dis[...]
    z = jax.nn.relu(z * d + b[...])
    sc = bw[...] * lax.rsqrt(brv[...] + BN_EPS)
    z = (z - brm[...]) * sc + bb[...]
    h2 = lax.dot_general(z, W2[...], (((1,), (1,)), ((), ())),
                         preferred_element_type=_f32)
    h2 = h2 * d
    h2lo_o[...] = h2[:, :HALF]
    h2hi_o[...] = h2[:, HALF:]


def _tc2_call(acclo, acchi, hlo, hhi, dis, b1, bn_w, bn_b, bn_rm, bn_rv, W2):
    row = lambda i: (i, 0)
    fixed = lambda i: (0, 0)
    return pl.pallas_call(
        _tc2_body,
        grid=(N // ROWB,),
        in_specs=[
            pl.BlockSpec((ROWB, HALF), row),
            pl.BlockSpec((ROWB, HALF), row),
            pl.BlockSpec((ROWB, HALF), row),
            pl.BlockSpec((ROWB, HALF), row),
            pl.BlockSpec((ROWB, 1), row),
            pl.BlockSpec((1, CL1), fixed),
            pl.BlockSpec((1, CL1), fixed),
            pl.BlockSpec((1, CL1), fixed),
            pl.BlockSpec((1, CL1), fixed),
            pl.BlockSpec((1, CL1), fixed),
            pl.BlockSpec((CL2, CL1), fixed),
        ],
        out_specs=[
            pl.BlockSpec((ROWB, HALF), row),
            pl.BlockSpec((ROWB, HALF), row),
        ],
        out_shape=[jax.ShapeDtypeStruct((N, HALF), _f32),
                   jax.ShapeDtypeStruct((N, HALF), _f32)],
    )(acclo, acchi, hlo, hhi, dis, b1, bn_w, bn_b, bn_rm, bn_rv, W2)


def _tc3_body(acclo, acchi, hlo, hhi, dis, b, bw, bb, brm, brv, lw, lb, out_o):
    z = jnp.concatenate([acclo[...] + hlo[...], acchi[...] + hhi[...]], axis=1)
    z = jax.nn.relu(z * dis[...] + b[...])
    sc = bw[...] * lax.rsqrt(brv[...] + BN_EPS)
    z = (z - brm[...]) * sc + bb[...]
    out = lax.dot_general(z, lw[...], (((1,), (1,)), ((), ())),
                          preferred_element_type=_f32)
    out_o[...] = out + lb[...]


def _tc3_call(acclo, acchi, hlo, hhi, dis, b2, bn_w, bn_b, bn_rm, bn_rv,
              lin_w, lin_b):
    row = lambda i: (i, 0)
    fixed = lambda i: (0, 0)
    return pl.pallas_call(
        _tc3_body,
        grid=(N // ROWB,),
        in_specs=[
            pl.BlockSpec((ROWB, HALF), row),
            pl.BlockSpec((ROWB, HALF), row),
            pl.BlockSpec((ROWB, HALF), row),
            pl.BlockSpec((ROWB, HALF), row),
            pl.BlockSpec((ROWB, 1), row),
            pl.BlockSpec((1, CL2), fixed),
            pl.BlockSpec((1, CL2), fixed),
            pl.BlockSpec((1, CL2), fixed),
            pl.BlockSpec((1, CL2), fixed),
            pl.BlockSpec((1, CL2), fixed),
            pl.BlockSpec((OUT_F, CL2), fixed),
            pl.BlockSpec((1, OUT_F), fixed),
        ],
        out_specs=pl.BlockSpec((ROWB, OUT_F), row),
        out_shape=jax.ShapeDtypeStruct((N, OUT_F), _f32),
    )(acclo, acchi, hlo, hhi, dis, b2, bn_w, bn_b, bn_rm, bn_rv, lin_w, lin_b)


def kernel(x, edge_index, edge_weight, W1, b1, W2, b2, lin_w, lin_b,
           bn1_w, bn1_b, bn1_rm, bn1_rv, bn2_w, bn2_b, bn2_rm, bn2_rv):
    pad = E_PAD - E
    src2 = jnp.concatenate(
        [edge_index[0].astype(jnp.int32), jnp.zeros((pad,), jnp.int32)]
    ).reshape(NCH_PAD, CHUNK)
    dst2 = jnp.concatenate(
        [edge_index[1].astype(jnp.int32), jnp.zeros((pad,), jnp.int32)]
    ).reshape(NCH_PAD, CHUNK)
    ew2 = jnp.concatenate(
        [edge_weight.astype(_f32), jnp.zeros((pad,), _f32)]
    ).reshape(NCH_PAD, CHUNK)

    deg_a, deg_b = _deg_call(dst2, ew2)
    dega = deg_a.reshape(N, 1)
    degb = deg_b.reshape(N, 1)

    dis, h1lo, h1hi = _tc1_call(dega, degb, x, W1)
    acc1lo, acc1hi = _prop_call(src2, dst2, ew2, h1lo, h1hi)
    acc1lo, acc1hi = acc1lo[:N], acc1hi[:N]
    h2lo, h2hi = _tc2_call(acc1lo, acc1hi, h1lo, h1hi, dis,
                           b1.reshape(1, CL1), bn1_w.reshape(1, CL1),
                           bn1_b.reshape(1, CL1), bn1_rm.reshape(1, CL1),
                           bn1_rv.reshape(1, CL1), W2)
    acc2lo, acc2hi = _prop_call(src2, dst2, ew2, h2lo, h2hi)
    acc2lo, acc2hi = acc2lo[:N], acc2hi[:N]
    out = _tc3_call(acc2lo, acc2hi, h2lo, h2hi, dis,
                    b2.reshape(1, CL2), bn2_w.reshape(1, CL2),
                    bn2_b.reshape(1, CL2), bn2_rm.reshape(1, CL2),
                    bn2_rv.reshape(1, CL2), lin_w, lin_b.reshape(1, OUT_F))
    return out
